# baseline (speedup 1.0000x reference)
import functools

import jax
import jax.numpy as jnp
from jax import lax
from jax.experimental import pallas as pl
from jax.experimental.pallas import tpu as pltpu
from jax.experimental.pallas import tpu_sc as plsc

B = 2
V = 4096
F_IN = 16
N_K = 16
N_DIM = 4
N_PROP = 16
N_FILTERS = 48
TILE = 1024


def _mlp_heads(g, W0, b0, W1, b1, W2, b2, Ws, bs_row, bs_col, Wf, bf):
    h = jnp.tanh(g @ W0 + b0)
    h = jnp.tanh(h @ W1 + b1)
    h = jnp.tanh(h @ W2 + b2)
    coords = h @ Ws + bs_row
    coordsT = lax.dot_general(Ws, h, (((0,), (1,)), ((), ()))) + bs_col
    feats = h @ Wf + bf
    return h, coords, coordsT, feats


def _gex(y):
    mean = jnp.mean(y, axis=0, keepdims=True)
    return jnp.concatenate([y, jnp.broadcast_to(mean, y.shape)], axis=1)


def _pre_body(x_ref, W0, b0, W1, b1, W2, b2, Ws, bs_row, bs_col, Wf, bf,
              h_ref, coords_ref, coordsT_ref, feats_ref):
    g = _gex(x_ref[0])
    h, coords, coordsT, feats = _mlp_heads(
        g, W0[...], b0[...], W1[...], b1[...], W2[...], b2[...],
        Ws[...], bs_row[...], bs_col[...], Wf[...], bf[...])
    h_ref[0] = h
    coords_ref[0] = coords
    coordsT_ref[0] = coordsT
    feats_ref[0] = feats


import numpy as np

_INT_MIN = np.int32(-(2 ** 31))


def _knn_body(coords_ref, coordsT_ref, idx_ref, w_ref):
    ct = coords_ref[...]
    cT = coordsT_ref[...]
    r_col = jnp.sum(ct * ct, axis=1, keepdims=True)
    r_row = jnp.sum(cT * cT, axis=0, keepdims=True)
    prod = lax.dot_general(ct, cT, (((1,), (0,)), ((), ())))
    d2 = r_col + r_row - 2.0 * prod
    colids = lax.broadcasted_iota(jnp.int32, (TILE, V), 1)
    pkb = (lax.bitcast_convert_type(d2, jnp.int32) & np.int32(-4096)) | colids
    pk = lax.bitcast_convert_type(pkb, jnp.float32)
    idx_cols, w_cols = [], []
    thr = None
    for k in range(N_K):
        cand = pk if k == 0 else jnp.where(pk > thr, pk, jnp.inf)
        m = jnp.min(cand, axis=1, keepdims=True)
        thr = m
        if k > 0:
            mbits = lax.bitcast_convert_type(m, jnp.int32)
            ik = mbits & 4095
            dist = lax.bitcast_convert_type(mbits & np.int32(-4096), jnp.float32)
            idx_cols.append(ik)
            w_cols.append(jnp.exp(-jnp.abs(dist * 10.0)))
    idx_cols.append(idx_cols[0] * 0)
    w_cols.append(w_cols[0] * 0)
    idx_ref[...] = jnp.concatenate(idx_cols, axis=1)
    w_ref[...] = jnp.concatenate(w_cols, axis=1)


_NW = 32
_VPW = V // _NW
_EPW = _VPW * N_K


_PASS_V = 128
_PASS_E = _PASS_V * N_K


def _sc_collect(table, idx_flat, w_flat):
    mesh = plsc.VectorSubcoreMesh(core_axis_name="c", subcore_axis_name="s")

    @functools.partial(
        pl.kernel,
        out_type=jax.ShapeDtypeStruct((V, 2 * N_PROP), jnp.float32),
        mesh=mesh,
        scratch_types=[
            pltpu.VMEM((_PASS_E,), jnp.int32),
            pltpu.VMEM((_PASS_E,), jnp.float32),
            pltpu.VMEM((_PASS_E, N_PROP), jnp.float32),
            pltpu.VMEM((_PASS_V, 2 * N_PROP), jnp.float32),
            pltpu.SemaphoreType.DMA,
        ],
        compiler_params=pltpu.CompilerParams(use_tc_tiling_on_sc=False),
    )
    def run(table_hbm, idx_hbm, w_hbm, out_hbm, idx_v, w_v, rows_v, out_v, sem):
        wid = lax.axis_index("s") * 2 + lax.axis_index("c")
        for ps in range(_VPW // _PASS_V):
            ebase = wid * _EPW + ps * _PASS_E
            pltpu.sync_copy(idx_hbm.at[pl.ds(ebase, _PASS_E)], idx_v)
            pltpu.sync_copy(w_hbm.at[pl.ds(ebase, _PASS_E)], w_v)
            copies = [
                pltpu.async_copy(
                    table_hbm.at[idx_v.at[pl.ds(c * 128, 128)]],
                    rows_v.at[pl.ds(c * 128, 128)], sem)
                for c in range(_PASS_E // 128)
            ]
            for cp in copies:
                cp.wait()

            def body(v, carry):
                base = v * N_K
                wrow = w_v[pl.ds(base, 16)]
                def wk(k):
                    return lax.gather(
                        wrow, jnp.full((16, 1), k, jnp.int32),
                        dimension_numbers=lax.GatherDimensionNumbers(
                            offset_dims=(), collapsed_slice_dims=(0,),
                            start_index_map=(0,)),
                        slice_sizes=(1,),
                        mode=lax.GatherScatterMode.PROMISE_IN_BOUNDS)
                p = rows_v[base, :] * wk(0)
                maxa, suma = p, p
                for k in range(1, N_K - 1):
                    p = rows_v[base + k, :] * wk(k)
                    maxa = jnp.maximum(maxa, p)
                    suma = suma + p
                out_v[v, pl.ds(0, N_PROP)] = maxa
                out_v[v, pl.ds(N_PROP, N_PROP)] = suma * (1.0 / (N_K - 1))
                return carry

            lax.fori_loop(0, _PASS_V, body, 0)
            pltpu.sync_copy(
                out_v, out_hbm.at[pl.ds(wid * _VPW + ps * _PASS_V, _PASS_V)])

    return run(table, idx_flat, w_flat)


def _block_out(h, coll, Wo, bo, scale, shift):
    upd = jnp.concatenate([h, coll], axis=1)
    return jnp.tanh(upd @ Wo + bo) * scale + shift


def _mid_body(h_ref, coll_ref, Wo, bo, scale, shift,
              W0, b0, W1, b1, W2, b2, Ws, bs_row, bs_col, Wf, bf,
              y_ref, h2_ref, coords_ref, coordsT_ref, feats_ref):
    y = _block_out(h_ref[0], coll_ref[0], Wo[...], bo[...], scale[...], shift[...])
    y_ref[0] = y
    g = _gex(y)
    h2, coords, coordsT, feats = _mlp_heads(
        g, W0[...], b0[...], W1[...], b1[...], W2[...], b2[...],
        Ws[...], bs_row[...], bs_col[...], Wf[...], bf[...])
    h2_ref[0] = h2
    coords_ref[0] = coords
    coordsT_ref[0] = coordsT
    feats_ref[0] = feats


def _final_body(h_ref, coll_ref, Wo, bo, scale, shift,
                y0_ref, y1_ref, y2_ref, Wq0, bq0, Wq1, bq1, z_ref):
    y3 = _block_out(h_ref[0], coll_ref[0], Wo[...], bo[...], scale[...], shift[...])
    z = jnp.concatenate([y0_ref[0], y1_ref[0], y2_ref[0], y3], axis=1)
    z = jnp.maximum(z @ Wq0[...] + bq0[...], 0.0)
    z = jax.nn.sigmoid(z @ Wq1[...] + bq1[...])
    z_ref[0] = z


def _full_spec(shape):
    n = len(shape)
    return pl.BlockSpec(shape, lambda *a: (0,) * n)


def _batch_spec(shape):
    return pl.BlockSpec((1,) + shape, lambda b: (b, 0, 0))


def _f32(shape):
    return jax.ShapeDtypeStruct(shape, jnp.float32)


def _blk_weights(blk):
    bn = blk["bn"]
    scale = bn["gamma"] / jnp.sqrt(bn["var"] + 1e-3)
    shift = bn["beta"] - bn["mean"] * scale
    return (
        blk["d0"]["W"], blk["d0"]["b"][None, :],
        blk["d1"]["W"], blk["d1"]["b"][None, :],
        blk["d2"]["W"], blk["d2"]["b"][None, :],
        blk["gn_s"]["W"], blk["gn_s"]["b"][None, :], blk["gn_s"]["b"][:, None],
        blk["gn_f"]["W"], blk["gn_f"]["b"][None, :],
    ), (blk["gn_o"]["W"], blk["gn_o"]["b"][None, :], scale[None, :], shift[None, :])


def _knn_call(coords, coordsT, feats):
    colls = []
    for b in range(B):
        idx, w = pl.pallas_call(
            _knn_body,
            grid=(V // TILE,),
            in_specs=[
                pl.BlockSpec((TILE, N_DIM), lambda t: (t, 0)),
                pl.BlockSpec((N_DIM, V), lambda t: (0, 0)),
            ],
            out_specs=[pl.BlockSpec((TILE, N_K), lambda t: (t, 0)),
                       pl.BlockSpec((TILE, N_K), lambda t: (t, 0))],
            out_shape=[jax.ShapeDtypeStruct((V, N_K), jnp.int32),
                       _f32((V, N_K))],
        )(coords[b], coordsT[b])
        colls.append(_sc_collect(feats[b], idx.reshape(V * N_K),
                                 w.reshape(V * N_K)))
    return jnp.stack(colls)


def kernel(x, params):
    blocks = params["blocks"]
    mlp_w, out_w = _blk_weights(blocks[0])
    mlp_specs = [_full_spec(w.shape) for w in mlp_w]

    h, coords, coordsT, feats = pl.pallas_call(
        _pre_body,
        grid=(B,),
        in_specs=[_batch_spec((V, F_IN))] + mlp_specs,
        out_specs=[_batch_spec((V, 32)), _batch_spec((V, N_DIM)),
                   _batch_spec((N_DIM, V)), _batch_spec((V, N_PROP))],
        out_shape=[_f32((B, V, 32)), _f32((B, V, N_DIM)),
                   _f32((B, N_DIM, V)), _f32((B, V, N_PROP))],
    )(x, *mlp_w)

    ys = []
    for i in range(3):
        coll = _knn_call(coords, coordsT, feats)
        next_mlp_w, _ = _blk_weights(blocks[i + 1])
        next_specs = [_full_spec(w.shape) for w in next_mlp_w]
        ow_specs = [_full_spec(w.shape) for w in out_w]
        y, h, coords, coordsT, feats = pl.pallas_call(
            _mid_body,
            grid=(B,),
            in_specs=[_batch_spec((V, 32)), _batch_spec((V, 2 * N_PROP))]
                     + ow_specs + next_specs,
            out_specs=[_batch_spec((V, N_FILTERS)), _batch_spec((V, 32)),
                       _batch_spec((V, N_DIM)), _batch_spec((N_DIM, V)),
                       _batch_spec((V, N_PROP))],
            out_shape=[_f32((B, V, N_FILTERS)), _f32((B, V, 32)),
                       _f32((B, V, N_DIM)), _f32((B, N_DIM, V)),
                       _f32((B, V, N_PROP))],
        )(h, coll, *out_w, *next_mlp_w)
        ys.append(y)
        _, out_w = _blk_weights(blocks[i + 1])

    coll = _knn_call(coords, coordsT, feats)
    q0W, q0b = params["out0"]["W"], params["out0"]["b"][None, :]
    q1W, q1b = params["out1"]["W"], params["out1"]["b"][None, :]
    ow_specs = [_full_spec(w.shape) for w in out_w]
    z = pl.pallas_call(
        _final_body,
        grid=(B,),
        in_specs=[_batch_spec((V, 32)), _batch_spec((V, 2 * N_PROP))]
                 + ow_specs
                 + [_batch_spec((V, N_FILTERS))] * 3
                 + [_full_spec(q0W.shape), _full_spec((1, 64)),
                    _full_spec(q1W.shape), _full_spec((1, 1))],
        out_specs=_batch_spec((V, 1)),
        out_shape=_f32((B, V, 1)),
    )(h, coll, *out_w, ys[0], ys[1], ys[2], q0W, q0b, q1W, q1b)
    return z


# speedup vs baseline: 1.1346x; 1.1346x over previous
import functools

import jax
import jax.numpy as jnp
from jax import lax
from jax.experimental import pallas as pl
from jax.experimental.pallas import tpu as pltpu
from jax.experimental.pallas import tpu_sc as plsc

B = 2
V = 4096
F_IN = 16
N_K = 16
N_DIM = 4
N_PROP = 16
N_FILTERS = 48
TILE = 512


def _mlp_heads(g, W0, b0, W1, b1, W2, b2, Ws, bs_row, bs_col, Wf, bf):
    h = jnp.tanh(g @ W0 + b0)
    h = jnp.tanh(h @ W1 + b1)
    h = jnp.tanh(h @ W2 + b2)
    coords = h @ Ws + bs_row
    coordsT = lax.dot_general(Ws, h, (((0,), (1,)), ((), ()))) + bs_col
    feats = h @ Wf + bf
    return h, coords, coordsT, feats


def _gex(y):
    mean = jnp.mean(y, axis=0, keepdims=True)
    return jnp.concatenate([y, jnp.broadcast_to(mean, y.shape)], axis=1)


def _pre_body(x_ref, W0, b0, W1, b1, W2, b2, Ws, bs_row, bs_col, Wf, bf,
              h_ref, coords_ref, coordsT_ref, feats_ref):
    g = _gex(x_ref[0])
    h, coords, coordsT, feats = _mlp_heads(
        g, W0[...], b0[...], W1[...], b1[...], W2[...], b2[...],
        Ws[...], bs_row[...], bs_col[...], Wf[...], bf[...])
    h_ref[0] = h
    coords_ref[0] = coords
    coordsT_ref[0] = coordsT
    feats_ref[0] = feats


import numpy as np

_INT_MIN = np.int32(-(2 ** 31))


def _knn_body(coords_ref, coordsT_ref, idx_ref, w_ref):
    ct = coords_ref[...]
    cT = coordsT_ref[...]
    r_col = jnp.sum(ct * ct, axis=1, keepdims=True)
    r_row = jnp.sum(cT * cT, axis=0, keepdims=True)
    prod = lax.dot_general(ct, cT, (((1,), (0,)), ((), ())))
    d2 = r_col + r_row - 2.0 * prod
    colids = lax.broadcasted_iota(jnp.int32, (TILE, V), 1)
    pkb = (lax.bitcast_convert_type(d2, jnp.int32) & np.int32(-4096)) | colids
    pk = lax.bitcast_convert_type(pkb, jnp.float32)
    idx_cols, w_cols = [], []
    thr = None
    for k in range(N_K):
        cand = pk if k == 0 else jnp.where(pk > thr, pk, jnp.inf)
        m = jnp.min(cand, axis=1, keepdims=True)
        thr = m
        if k > 0:
            mbits = lax.bitcast_convert_type(m, jnp.int32)
            ik = mbits & 4095
            dist = lax.bitcast_convert_type(mbits & np.int32(-4096), jnp.float32)
            idx_cols.append(ik)
            w_cols.append(jnp.exp(-jnp.abs(dist * 10.0)))
    idx_cols.append(idx_cols[0] * 0)
    w_cols.append(w_cols[0] * 0)
    idx_ref[...] = jnp.concatenate(idx_cols, axis=1)
    w_ref[...] = jnp.concatenate(w_cols, axis=1)


_NW = 32
_VPW = V // _NW
_EPW = _VPW * N_K


_PASS_V = 128
_PASS_E = _PASS_V * N_K


def _sc_collect(table, idx_flat, w_flat):
    mesh = plsc.VectorSubcoreMesh(core_axis_name="c", subcore_axis_name="s")

    @functools.partial(
        pl.kernel,
        out_type=jax.ShapeDtypeStruct((V, 2 * N_PROP), jnp.float32),
        mesh=mesh,
        scratch_types=[
            pltpu.VMEM((_PASS_E,), jnp.int32),
            pltpu.VMEM((_PASS_E,), jnp.float32),
            pltpu.VMEM((_PASS_E, N_PROP), jnp.float32),
            pltpu.VMEM((_PASS_V, 2 * N_PROP), jnp.float32),
            pltpu.SemaphoreType.DMA,
        ],
        compiler_params=pltpu.CompilerParams(use_tc_tiling_on_sc=False),
    )
    def run(table_hbm, idx_hbm, w_hbm, out_hbm, idx_v, w_v, rows_v, out_v, sem):
        wid = lax.axis_index("s") * 2 + lax.axis_index("c")
        for ps in range(_VPW // _PASS_V):
            ebase = wid * _EPW + ps * _PASS_E
            pltpu.sync_copy(idx_hbm.at[pl.ds(ebase, _PASS_E)], idx_v)
            pltpu.sync_copy(w_hbm.at[pl.ds(ebase, _PASS_E)], w_v)
            copies = [
                pltpu.async_copy(
                    table_hbm.at[idx_v.at[pl.ds(c * 128, 128)]],
                    rows_v.at[pl.ds(c * 128, 128)], sem)
                for c in range(_PASS_E // 128)
            ]
            for cp in copies:
                cp.wait()

            def body(v, carry):
                base = v * N_K
                wrow = w_v[pl.ds(base, 16)]
                def wk(k):
                    return lax.gather(
                        wrow, jnp.full((16, 1), k, jnp.int32),
                        dimension_numbers=lax.GatherDimensionNumbers(
                            offset_dims=(), collapsed_slice_dims=(0,),
                            start_index_map=(0,)),
                        slice_sizes=(1,),
                        mode=lax.GatherScatterMode.PROMISE_IN_BOUNDS)
                p = rows_v[base, :] * wk(0)
                maxa, suma = p, p
                for k in range(1, N_K - 1):
                    p = rows_v[base + k, :] * wk(k)
                    maxa = jnp.maximum(maxa, p)
                    suma = suma + p
                out_v[v, pl.ds(0, N_PROP)] = maxa
                out_v[v, pl.ds(N_PROP, N_PROP)] = suma * (1.0 / (N_K - 1))
                return carry

            lax.fori_loop(0, _PASS_V, body, 0)
            pltpu.sync_copy(
                out_v, out_hbm.at[pl.ds(wid * _VPW + ps * _PASS_V, _PASS_V)])

    return run(table, idx_flat, w_flat)


def _block_out(h, coll, Wo, bo, scale, shift):
    upd = jnp.concatenate([h, coll], axis=1)
    return jnp.tanh(upd @ Wo + bo) * scale + shift


def _mid_body(h_ref, coll_ref, Wo, bo, scale, shift,
              W0, b0, W1, b1, W2, b2, Ws, bs_row, bs_col, Wf, bf,
              y_ref, h2_ref, coords_ref, coordsT_ref, feats_ref):
    y = _block_out(h_ref[0], coll_ref[0], Wo[...], bo[...], scale[...], shift[...])
    y_ref[0] = y
    g = _gex(y)
    h2, coords, coordsT, feats = _mlp_heads(
        g, W0[...], b0[...], W1[...], b1[...], W2[...], b2[...],
        Ws[...], bs_row[...], bs_col[...], Wf[...], bf[...])
    h2_ref[0] = h2
    coords_ref[0] = coords
    coordsT_ref[0] = coordsT
    feats_ref[0] = feats


def _final_body(h_ref, coll_ref, Wo, bo, scale, shift,
                y0_ref, y1_ref, y2_ref, Wq0, bq0, Wq1, bq1, z_ref):
    y3 = _block_out(h_ref[0], coll_ref[0], Wo[...], bo[...], scale[...], shift[...])
    z = jnp.concatenate([y0_ref[0], y1_ref[0], y2_ref[0], y3], axis=1)
    z = jnp.maximum(z @ Wq0[...] + bq0[...], 0.0)
    z = jax.nn.sigmoid(z @ Wq1[...] + bq1[...])
    z_ref[0] = z


def _full_spec(shape):
    n = len(shape)
    return pl.BlockSpec(shape, lambda *a: (0,) * n)


def _batch_spec(shape):
    return pl.BlockSpec((1,) + shape, lambda b: (b, 0, 0))


def _f32(shape):
    return jax.ShapeDtypeStruct(shape, jnp.float32)


def _blk_weights(blk):
    bn = blk["bn"]
    scale = bn["gamma"] / jnp.sqrt(bn["var"] + 1e-3)
    shift = bn["beta"] - bn["mean"] * scale
    return (
        blk["d0"]["W"], blk["d0"]["b"][None, :],
        blk["d1"]["W"], blk["d1"]["b"][None, :],
        blk["d2"]["W"], blk["d2"]["b"][None, :],
        blk["gn_s"]["W"], blk["gn_s"]["b"][None, :], blk["gn_s"]["b"][:, None],
        blk["gn_f"]["W"], blk["gn_f"]["b"][None, :],
    ), (blk["gn_o"]["W"], blk["gn_o"]["b"][None, :], scale[None, :], shift[None, :])


def _knn_call(coords, coordsT, feats):
    colls = []
    for b in range(B):
        idx, w = pl.pallas_call(
            _knn_body,
            grid=(V // TILE,),
            in_specs=[
                pl.BlockSpec((TILE, N_DIM), lambda t: (t, 0)),
                pl.BlockSpec((N_DIM, V), lambda t: (0, 0)),
            ],
            out_specs=[pl.BlockSpec((TILE, N_K), lambda t: (t, 0)),
                       pl.BlockSpec((TILE, N_K), lambda t: (t, 0))],
            out_shape=[jax.ShapeDtypeStruct((V, N_K), jnp.int32),
                       _f32((V, N_K))],
        )(coords[b], coordsT[b])
        colls.append(_sc_collect(feats[b], idx.reshape(V * N_K),
                                 w.reshape(V * N_K)))
    return jnp.stack(colls)


def kernel(x, params):
    blocks = params["blocks"]
    mlp_w, out_w = _blk_weights(blocks[0])
    mlp_specs = [_full_spec(w.shape) for w in mlp_w]

    h, coords, coordsT, feats = pl.pallas_call(
        _pre_body,
        grid=(B,),
        in_specs=[_batch_spec((V, F_IN))] + mlp_specs,
        out_specs=[_batch_spec((V, 32)), _batch_spec((V, N_DIM)),
                   _batch_spec((N_DIM, V)), _batch_spec((V, N_PROP))],
        out_shape=[_f32((B, V, 32)), _f32((B, V, N_DIM)),
                   _f32((B, N_DIM, V)), _f32((B, V, N_PROP))],
    )(x, *mlp_w)

    ys = []
    for i in range(3):
        coll = _knn_call(coords, coordsT, feats)
        next_mlp_w, _ = _blk_weights(blocks[i + 1])
        next_specs = [_full_spec(w.shape) for w in next_mlp_w]
        ow_specs = [_full_spec(w.shape) for w in out_w]
        y, h, coords, coordsT, feats = pl.pallas_call(
            _mid_body,
            grid=(B,),
            in_specs=[_batch_spec((V, 32)), _batch_spec((V, 2 * N_PROP))]
                     + ow_specs + next_specs,
            out_specs=[_batch_spec((V, N_FILTERS)), _batch_spec((V, 32)),
                       _batch_spec((V, N_DIM)), _batch_spec((N_DIM, V)),
                       _batch_spec((V, N_PROP))],
            out_shape=[_f32((B, V, N_FILTERS)), _f32((B, V, 32)),
                       _f32((B, V, N_DIM)), _f32((B, N_DIM, V)),
                       _f32((B, V, N_PROP))],
        )(h, coll, *out_w, *next_mlp_w)
        ys.append(y)
        _, out_w = _blk_weights(blocks[i + 1])

    coll = _knn_call(coords, coordsT, feats)
    q0W, q0b = params["out0"]["W"], params["out0"]["b"][None, :]
    q1W, q1b = params["out1"]["W"], params["out1"]["b"][None, :]
    ow_specs = [_full_spec(w.shape) for w in out_w]
    z = pl.pallas_call(
        _final_body,
        grid=(B,),
        in_specs=[_batch_spec((V, 32)), _batch_spec((V, 2 * N_PROP))]
                 + ow_specs
                 + [_batch_spec((V, N_FILTERS))] * 3
                 + [_full_spec(q0W.shape), _full_spec((1, 64)),
                    _full_spec(q1W.shape), _full_spec((1, 1))],
        out_specs=_batch_spec((V, 1)),
        out_shape=_f32((B, V, 1)),
    )(h, coll, *out_w, ys[0], ys[1], ys[2], q0W, q0b, q1W, q1b)
    return z


# speedup vs baseline: 1.1455x; 1.0096x over previous
import functools

import jax
import jax.numpy as jnp
from jax import lax
from jax.experimental import pallas as pl
from jax.experimental.pallas import tpu as pltpu
from jax.experimental.pallas import tpu_sc as plsc

B = 2
V = 4096
F_IN = 16
N_K = 16
N_DIM = 4
N_PROP = 16
N_FILTERS = 48
TILE = 512


def _mlp_heads(g, W0, b0, W1, b1, W2, b2, Ws, bs_row, bs_col, Wf, bf):
    h = jnp.tanh(g @ W0 + b0)
    h = jnp.tanh(h @ W1 + b1)
    h = jnp.tanh(h @ W2 + b2)
    coords = h @ Ws + bs_row
    coordsT = lax.dot_general(Ws, h, (((0,), (1,)), ((), ()))) + bs_col
    feats = h @ Wf + bf
    return h, coords, coordsT, feats


def _gex(y):
    mean = jnp.mean(y, axis=0, keepdims=True)
    return jnp.concatenate([y, jnp.broadcast_to(mean, y.shape)], axis=1)


def _pre_body(x_ref, W0, b0, W1, b1, W2, b2, Ws, bs_row, bs_col, Wf, bf,
              h_ref, coords_ref, coordsT_ref, feats_ref):
    g = _gex(x_ref[0])
    h, coords, coordsT, feats = _mlp_heads(
        g, W0[...], b0[...], W1[...], b1[...], W2[...], b2[...],
        Ws[...], bs_row[...], bs_col[...], Wf[...], bf[...])
    h_ref[0] = h
    coords_ref[0] = coords
    coordsT_ref[0] = coordsT
    feats_ref[0] = feats


import numpy as np

_INT_MIN = np.int32(-(2 ** 31))


def _knn_body(coords_ref, coordsT_ref, idx_ref, w_ref):
    ct = coords_ref[...]
    cT = coordsT_ref[...]
    r_col = jnp.sum(ct * ct, axis=1, keepdims=True)
    r_row = jnp.sum(cT * cT, axis=0, keepdims=True)
    prod = lax.dot_general(ct, cT, (((1,), (0,)), ((), ())))
    d2 = r_col + r_row - 2.0 * prod
    colids = lax.broadcasted_iota(jnp.int32, (TILE, V), 1)
    pkb = (lax.bitcast_convert_type(d2, jnp.int32) & np.int32(-4096)) | colids
    pk = lax.bitcast_convert_type(pkb, jnp.float32)
    idx_cols, w_cols = [], []
    thr = None
    for k in range(N_K):
        cand = pk if k == 0 else jnp.where(pk > thr, pk, jnp.inf)
        m = jnp.min(cand, axis=1, keepdims=True)
        thr = m
        if k > 0:
            mbits = lax.bitcast_convert_type(m, jnp.int32)
            ik = mbits & 4095
            dist = lax.bitcast_convert_type(mbits & np.int32(-4096), jnp.float32)
            idx_cols.append(ik)
            w_cols.append(jnp.exp(-jnp.abs(dist * 10.0)))
    idx_cols.append(idx_cols[0] * 0)
    w_cols.append(w_cols[0] * 0)
    idx_ref[...] = jnp.concatenate(idx_cols, axis=1)
    w_ref[...] = jnp.concatenate(w_cols, axis=1)


_NW = 32
_VPW = V // _NW
_EPW = _VPW * N_K


_PASS_V = 128
_PASS_E = _PASS_V * N_K


def _sc_collect(table, idx_flat, w_flat):
    mesh = plsc.VectorSubcoreMesh(core_axis_name="c", subcore_axis_name="s")

    @functools.partial(
        pl.kernel,
        out_type=jax.ShapeDtypeStruct((V, 2 * N_PROP), jnp.float32),
        mesh=mesh,
        scratch_types=[
            pltpu.VMEM((_PASS_E,), jnp.int32),
            pltpu.VMEM((_PASS_E,), jnp.float32),
            pltpu.VMEM((_PASS_E, N_PROP), jnp.float32),
            pltpu.VMEM((_PASS_V, 2 * N_PROP), jnp.float32),
            pltpu.SemaphoreType.DMA,
        ],
        compiler_params=pltpu.CompilerParams(use_tc_tiling_on_sc=False),
    )
    def run(table_hbm, idx_hbm, w_hbm, out_hbm, idx_v, w_v, rows_v, out_v, sem):
        wid = lax.axis_index("s") * 2 + lax.axis_index("c")
        for ps in range(_VPW // _PASS_V):
            ebase = wid * _EPW + ps * _PASS_E
            pltpu.sync_copy(idx_hbm.at[pl.ds(ebase, _PASS_E)], idx_v)
            pltpu.sync_copy(w_hbm.at[pl.ds(ebase, _PASS_E)], w_v)
            copies = [
                pltpu.async_copy(
                    table_hbm.at[idx_v.at[pl.ds(c * 128, 128)]],
                    rows_v.at[pl.ds(c * 128, 128)], sem)
                for c in range(_PASS_E // 128)
            ]
            for cp in copies:
                cp.wait()

            @functools.partial(plsc.parallel_loop, 0, _PASS_V, unroll=2)
            def body(v):
                base = v * N_K
                wrow = w_v[pl.ds(base, 16)]
                def wk(k):
                    return lax.gather(
                        wrow, jnp.full((16, 1), k, jnp.int32),
                        dimension_numbers=lax.GatherDimensionNumbers(
                            offset_dims=(), collapsed_slice_dims=(0,),
                            start_index_map=(0,)),
                        slice_sizes=(1,),
                        mode=lax.GatherScatterMode.PROMISE_IN_BOUNDS)
                p = rows_v[base, :] * wk(0)
                maxa, suma = p, p
                for k in range(1, N_K - 1):
                    p = rows_v[base + k, :] * wk(k)
                    maxa = jnp.maximum(maxa, p)
                    suma = suma + p
                out_v[v, pl.ds(0, N_PROP)] = maxa
                out_v[v, pl.ds(N_PROP, N_PROP)] = suma * (1.0 / (N_K - 1))

            pltpu.sync_copy(
                out_v, out_hbm.at[pl.ds(wid * _VPW + ps * _PASS_V, _PASS_V)])

    return run(table, idx_flat, w_flat)


def _block_out(h, coll, Wo, bo, scale, shift):
    upd = jnp.concatenate([h, coll], axis=1)
    return jnp.tanh(upd @ Wo + bo) * scale + shift


def _mid_body(h_ref, coll_ref, Wo, bo, scale, shift,
              W0, b0, W1, b1, W2, b2, Ws, bs_row, bs_col, Wf, bf,
              y_ref, h2_ref, coords_ref, coordsT_ref, feats_ref):
    y = _block_out(h_ref[0], coll_ref[0], Wo[...], bo[...], scale[...], shift[...])
    y_ref[0] = y
    g = _gex(y)
    h2, coords, coordsT, feats = _mlp_heads(
        g, W0[...], b0[...], W1[...], b1[...], W2[...], b2[...],
        Ws[...], bs_row[...], bs_col[...], Wf[...], bf[...])
    h2_ref[0] = h2
    coords_ref[0] = coords
    coordsT_ref[0] = coordsT
    feats_ref[0] = feats


def _final_body(h_ref, coll_ref, Wo, bo, scale, shift,
                y0_ref, y1_ref, y2_ref, Wq0, bq0, Wq1, bq1, z_ref):
    y3 = _block_out(h_ref[0], coll_ref[0], Wo[...], bo[...], scale[...], shift[...])
    z = jnp.concatenate([y0_ref[0], y1_ref[0], y2_ref[0], y3], axis=1)
    z = jnp.maximum(z @ Wq0[...] + bq0[...], 0.0)
    z = jax.nn.sigmoid(z @ Wq1[...] + bq1[...])
    z_ref[0] = z


def _full_spec(shape):
    n = len(shape)
    return pl.BlockSpec(shape, lambda *a: (0,) * n)


def _batch_spec(shape):
    return pl.BlockSpec((1,) + shape, lambda b: (b, 0, 0))


def _f32(shape):
    return jax.ShapeDtypeStruct(shape, jnp.float32)


def _blk_weights(blk):
    bn = blk["bn"]
    scale = bn["gamma"] / jnp.sqrt(bn["var"] + 1e-3)
    shift = bn["beta"] - bn["mean"] * scale
    return (
        blk["d0"]["W"], blk["d0"]["b"][None, :],
        blk["d1"]["W"], blk["d1"]["b"][None, :],
        blk["d2"]["W"], blk["d2"]["b"][None, :],
        blk["gn_s"]["W"], blk["gn_s"]["b"][None, :], blk["gn_s"]["b"][:, None],
        blk["gn_f"]["W"], blk["gn_f"]["b"][None, :],
    ), (blk["gn_o"]["W"], blk["gn_o"]["b"][None, :], scale[None, :], shift[None, :])


def _knn_call(coords, coordsT, feats):
    colls = []
    for b in range(B):
        idx, w = pl.pallas_call(
            _knn_body,
            grid=(V // TILE,),
            in_specs=[
                pl.BlockSpec((TILE, N_DIM), lambda t: (t, 0)),
                pl.BlockSpec((N_DIM, V), lambda t: (0, 0)),
            ],
            out_specs=[pl.BlockSpec((TILE, N_K), lambda t: (t, 0)),
                       pl.BlockSpec((TILE, N_K), lambda t: (t, 0))],
            out_shape=[jax.ShapeDtypeStruct((V, N_K), jnp.int32),
                       _f32((V, N_K))],
        )(coords[b], coordsT[b])
        colls.append(_sc_collect(feats[b], idx.reshape(V * N_K),
                                 w.reshape(V * N_K)))
    return jnp.stack(colls)


def kernel(x, params):
    blocks = params["blocks"]
    mlp_w, out_w = _blk_weights(blocks[0])
    mlp_specs = [_full_spec(w.shape) for w in mlp_w]

    h, coords, coordsT, feats = pl.pallas_call(
        _pre_body,
        grid=(B,),
        in_specs=[_batch_spec((V, F_IN))] + mlp_specs,
        out_specs=[_batch_spec((V, 32)), _batch_spec((V, N_DIM)),
                   _batch_spec((N_DIM, V)), _batch_spec((V, N_PROP))],
        out_shape=[_f32((B, V, 32)), _f32((B, V, N_DIM)),
                   _f32((B, N_DIM, V)), _f32((B, V, N_PROP))],
    )(x, *mlp_w)

    ys = []
    for i in range(3):
        coll = _knn_call(coords, coordsT, feats)
        next_mlp_w, _ = _blk_weights(blocks[i + 1])
        next_specs = [_full_spec(w.shape) for w in next_mlp_w]
        ow_specs = [_full_spec(w.shape) for w in out_w]
        y, h, coords, coordsT, feats = pl.pallas_call(
            _mid_body,
            grid=(B,),
            in_specs=[_batch_spec((V, 32)), _batch_spec((V, 2 * N_PROP))]
                     + ow_specs + next_specs,
            out_specs=[_batch_spec((V, N_FILTERS)), _batch_spec((V, 32)),
                       _batch_spec((V, N_DIM)), _batch_spec((N_DIM, V)),
                       _batch_spec((V, N_PROP))],
            out_shape=[_f32((B, V, N_FILTERS)), _f32((B, V, 32)),
                       _f32((B, V, N_DIM)), _f32((B, N_DIM, V)),
                       _f32((B, V, N_PROP))],
        )(h, coll, *out_w, *next_mlp_w)
        ys.append(y)
        _, out_w = _blk_weights(blocks[i + 1])

    coll = _knn_call(coords, coordsT, feats)
    q0W, q0b = params["out0"]["W"], params["out0"]["b"][None, :]
    q1W, q1b = params["out1"]["W"], params["out1"]["b"][None, :]
    ow_specs = [_full_spec(w.shape) for w in out_w]
    z = pl.pallas_call(
        _final_body,
        grid=(B,),
        in_specs=[_batch_spec((V, 32)), _batch_spec((V, 2 * N_PROP))]
                 + ow_specs
                 + [_batch_spec((V, N_FILTERS))] * 3
                 + [_full_spec(q0W.shape), _full_spec((1, 64)),
                    _full_spec(q1W.shape), _full_spec((1, 1))],
        out_specs=_batch_spec((V, 1)),
        out_shape=_f32((B, V, 1)),
    )(h, coll, *out_w, ys[0], ys[1], ys[2], q0W, q0b, q1W, q1b)
    return z
